# Initial kernel scaffold; baseline (speedup 1.0000x reference)
import jax
import jax.numpy as jnp
from jax.experimental import pallas as pl


def kernel(x, edge_index, batch, W1, b1, W2, b2):
    raise NotImplementedError("write your pallas kernel here")


# speedup vs baseline: 21.3955x; 21.3955x over previous
import functools
import jax
import jax.numpy as jnp
from jax import lax
from jax.experimental import pallas as pl
from jax.experimental.pallas import tpu as pltpu
from jax.experimental.pallas import tpu_sc as plsc

N = 50000
NPAD = 50048
E = 800000
DIN = 1433
H = 16
C = 7
G = 64
NC = 2
NS = 16
NW = NC * NS
CH = 128
NCHUNK = 200
EW = CH * NCHUNK
EPAD = EW * NW
BM = 1000
NBLK = N // BM
RPT = NPAD // NS
ZCH = 136
NZB = RPT // ZCH

_sc_mesh = plsc.VectorSubcoreMesh(core_axis_name="c", subcore_axis_name="s")


@functools.partial(
    pl.kernel,
    out_type=jax.ShapeDtypeStruct((NC * NPAD,), jnp.float32),
    mesh=_sc_mesh,
    scratch_types=[
        pltpu.VMEM((NCHUNK, CH), jnp.int32),
        pltpu.VMEM((CH,), jnp.float32),
        pltpu.VMEM((RPT,), jnp.float32),
        pltpu.VMEM_SHARED((NPAD,), jnp.float32),
    ],
    compiler_params=pltpu.CompilerParams(use_tc_tiling_on_sc=False),
)
def _sc_deg(dst_hbm, zeros_hbm, ones_hbm, out_hbm, dst_v, ones_v, stage_v,
            acc_sh):
    c = lax.axis_index("c")
    s = lax.axis_index("s")
    w = c * NS + s
    r0 = s * RPT
    pltpu.sync_copy(zeros_hbm.at[pl.ds(r0, RPT)], stage_v)
    pltpu.sync_copy(stage_v, acc_sh.at[pl.ds(r0, RPT)])
    pltpu.sync_copy(ones_hbm, ones_v)
    pltpu.sync_copy(dst_hbm.at[pl.ds(w * NCHUNK, NCHUNK)], dst_v)
    plsc.subcore_barrier()

    def body(j, carry):
        pltpu.sync_copy(ones_v, acc_sh.at[dst_v.at[j]], add=True)
        return carry

    lax.fori_loop(0, NCHUNK, body, 0)
    plsc.subcore_barrier()
    pltpu.sync_copy(acc_sh.at[pl.ds(r0, RPT)], stage_v)
    pltpu.sync_copy(stage_v, out_hbm.at[pl.ds(c * NPAD + r0, RPT)])


def _make_agg(F):
    def body_fn(y_hbm, src_hbm, dst_hbm, zeros_hbm, out_hbm,
                src_v, dst_v, rows_v, zbuf_v, acc_sh, gsem):
        c = lax.axis_index("c")
        s = lax.axis_index("s")
        w = c * NS + s
        r0 = s * RPT
        pltpu.sync_copy(zeros_hbm.at[pl.ds(0, ZCH)], zbuf_v)

        def zbody(k, carry):
            pltpu.sync_copy(zbuf_v, acc_sh.at[pl.ds(r0 + k * ZCH, ZCH)])
            return carry

        lax.fori_loop(0, NZB, zbody, 0)
        pltpu.sync_copy(src_hbm.at[pl.ds(w * NCHUNK, NCHUNK)], src_v)
        pltpu.sync_copy(dst_hbm.at[pl.ds(w * NCHUNK, NCHUNK)], dst_v)
        plsc.subcore_barrier()

        def body(j, carry):
            pltpu.async_copy(y_hbm.at[src_v.at[j]], rows_v, gsem).wait()
            pltpu.sync_copy(rows_v, acc_sh.at[dst_v.at[j]], add=True)
            return carry

        lax.fori_loop(0, NCHUNK, body, 0)
        plsc.subcore_barrier()

        def wbody(k, carry):
            pltpu.sync_copy(acc_sh.at[pl.ds(r0 + k * ZCH, ZCH)], zbuf_v)
            pltpu.sync_copy(zbuf_v, out_hbm.at[pl.ds(c * NPAD + r0 + k * ZCH, ZCH)])
            return carry

        lax.fori_loop(0, NZB, wbody, 0)

    return pl.kernel(
        body_fn,
        out_type=jax.ShapeDtypeStruct((NC * NPAD, F), jnp.float32),
        mesh=_sc_mesh,
        scratch_types=[
            pltpu.VMEM((NCHUNK, CH), jnp.int32),
            pltpu.VMEM((NCHUNK, CH), jnp.int32),
            pltpu.VMEM((CH, F), jnp.float32),
            pltpu.VMEM((ZCH, F), jnp.float32),
            pltpu.VMEM_SHARED((NPAD, F), jnp.float32),
            pltpu.SemaphoreType.DMA,
        ],
        compiler_params=pltpu.CompilerParams(use_tc_tiling_on_sc=False),
    )


_sc_agg16 = _make_agg(H)
_sc_agg8 = _make_agg(8)


def _dis_of(deg_block):
    return lax.rsqrt(deg_block[:, 0] + deg_block[:, 1] + 1.0)


def _mm_body(x_ref, w_ref, deg_ref, y_ref):
    dis = _dis_of(deg_ref[...])
    xw = jnp.dot(x_ref[...], w_ref[...], preferred_element_type=jnp.float32)
    y_ref[...] = xw * dis[:, None]


def _mid_body(s1a_ref, s1b_ref, y1_ref, deg_ref, b1_ref, w2_ref, y2_ref):
    dis = _dis_of(deg_ref[...])
    pre = dis[:, None] * (s1a_ref[...] + s1b_ref[...] + y1_ref[...]) + b1_ref[...]
    h1 = jnp.maximum(pre, 0.0)
    z2 = jnp.dot(h1, w2_ref[...], preferred_element_type=jnp.float32)
    y2_ref[...] = z2 * dis[:, None]


def _fin_body(s2a_ref, s2b_ref, y2_ref, deg_ref, b2_ref, batch_ref,
              ls_ref, ht_ref):
    i = pl.program_id(0)
    dis = _dis_of(deg_ref[...])
    x2 = dis[:, None] * (s2a_ref[...] + s2b_ref[...] + y2_ref[...]) + b2_ref[...]
    neg = jnp.float32(-jnp.inf)
    col = lax.broadcasted_iota(jnp.int32, (BM, 8), 1)
    valid = col < C
    x2m = jnp.where(valid, x2, neg)
    m = jnp.max(x2m, axis=1, keepdims=True)
    ex = jnp.where(valid, jnp.exp(x2 - m), 0.0)
    lse = jnp.log(jnp.sum(ex, axis=1, keepdims=True)) + m
    ls_ref[...] = lax.slice(x2 - lse, (0, 0), (BM, C))

    b = batch_ref[...]
    gid = lax.broadcasted_iota(jnp.int32, (BM, G), 1)
    mask = b == gid

    @pl.when(i == 0)
    def _():
        ht_ref[...] = jnp.full((C, G), neg, jnp.float32)

    rows = []
    for cc in range(C):
        xc = lax.slice(x2, (0, cc), (BM, cc + 1))
        rows.append(jnp.max(jnp.where(mask, xc, neg), axis=0, keepdims=True))
    ht_ref[...] = jnp.maximum(ht_ref[...], jnp.concatenate(rows, axis=0))


_row_spec = pl.BlockSpec((BM, H), lambda i: (i, 0))
_deg_spec = pl.BlockSpec((BM, 2), lambda i: (i, 0))

_tc_mm = pl.pallas_call(
    _mm_body,
    grid=(NBLK,),
    in_specs=[
        pl.BlockSpec((BM, DIN), lambda i: (i, 0)),
        pl.BlockSpec((DIN, H), lambda i: (0, 0)),
        _deg_spec,
    ],
    out_specs=_row_spec,
    out_shape=jax.ShapeDtypeStruct((N, H), jnp.float32),
)

_row8_spec = pl.BlockSpec((BM, 8), lambda i: (i, 0))

_tc_mid = pl.pallas_call(
    _mid_body,
    grid=(NBLK,),
    in_specs=[
        _row_spec, _row_spec, _row_spec, _deg_spec,
        pl.BlockSpec((1, H), lambda i: (0, 0)),
        pl.BlockSpec((H, 8), lambda i: (0, 0)),
    ],
    out_specs=_row8_spec,
    out_shape=jax.ShapeDtypeStruct((N, 8), jnp.float32),
)

_tc_fin = pl.pallas_call(
    _fin_body,
    grid=(NBLK,),
    in_specs=[
        _row8_spec, _row8_spec, _row8_spec, _deg_spec,
        pl.BlockSpec((1, 8), lambda i: (0, 0)),
        pl.BlockSpec((BM, 1), lambda i: (i, 0)),
    ],
    out_specs=(
        pl.BlockSpec((BM, C), lambda i: (i, 0)),
        pl.BlockSpec((C, G), lambda i: (0, 0)),
    ),
    out_shape=(
        jax.ShapeDtypeStruct((N, C), jnp.float32),
        jax.ShapeDtypeStruct((C, G), jnp.float32),
    ),
)


def kernel(x, edge_index, batch, W1, b1, W2, b2):
    ei = edge_index.astype(jnp.int32)
    npad = EPAD - E
    ar = jnp.arange(npad, dtype=jnp.int32)
    pad_src = (ar * 97) % N
    pad_dst = N + (ar % (NPAD - N))
    srcp = jnp.concatenate([ei[0], pad_src]).reshape(NW * NCHUNK, CH)
    dstp = jnp.concatenate([ei[1], pad_dst]).reshape(NW * NCHUNK, CH)

    zeros16 = jnp.zeros((NPAD, H), jnp.float32)
    zeros8 = jnp.zeros((NPAD, 8), jnp.float32)
    zeros1 = jnp.zeros((NPAD,), jnp.float32)
    ones = jnp.ones((CH,), jnp.float32)

    deg2 = _sc_deg(dstp, zeros1, ones)
    degT = deg2.reshape(NC, NPAD).T

    y1 = _tc_mm(x, W1, degT[:N])
    s1 = _sc_agg16(y1, srcp, dstp, zeros16)

    W2p = jnp.zeros((H, 8), jnp.float32).at[:, :C].set(W2)
    b1r = b1.reshape(1, H)
    y2 = _tc_mid(s1[:N], s1[NPAD:NPAD + N], y1, degT[:N], b1r, W2p)

    s2 = _sc_agg8(y2, srcp, dstp, zeros8)
    b2p = jnp.pad(b2, (0, 8 - C)).reshape(1, 8)
    batch2 = batch.astype(jnp.int32).reshape(N, 1)
    ls, ht = _tc_fin(s2[:N], s2[NPAD:NPAD + N], y2, degT[:N], b2p, batch2)
    return (ht.T, ls)


# speedup vs baseline: 24.9910x; 1.1680x over previous
import functools
import jax
import jax.numpy as jnp
from jax import lax
from jax.experimental import pallas as pl
from jax.experimental.pallas import tpu as pltpu
from jax.experimental.pallas import tpu_sc as plsc

N = 50000
NPAD = 50048
E = 800000
DIN = 1433
H = 16
C = 7
G = 64
NC = 2
NS = 16
NW = NC * NS
CH = 128
NCHUNK = 200
EW = CH * NCHUNK
EPAD = EW * NW
BM = 1000
NBLK = N // BM
RPT = NPAD // NS
ZCH = 136
NZB = RPT // ZCH

_sc_mesh = plsc.VectorSubcoreMesh(core_axis_name="c", subcore_axis_name="s")


@functools.partial(
    pl.kernel,
    out_type=jax.ShapeDtypeStruct((NC * NPAD,), jnp.float32),
    mesh=_sc_mesh,
    scratch_types=[
        pltpu.VMEM((NCHUNK, CH), jnp.int32),
        pltpu.VMEM((CH,), jnp.float32),
        pltpu.VMEM((RPT,), jnp.float32),
        pltpu.VMEM_SHARED((NPAD,), jnp.float32),
    ],
    compiler_params=pltpu.CompilerParams(use_tc_tiling_on_sc=False),
)
def _sc_deg(dst_hbm, zeros_hbm, ones_hbm, out_hbm, dst_v, ones_v, stage_v,
            acc_sh):
    c = lax.axis_index("c")
    s = lax.axis_index("s")
    w = c * NS + s
    r0 = s * RPT
    pltpu.sync_copy(zeros_hbm.at[pl.ds(r0, RPT)], stage_v)
    pltpu.sync_copy(stage_v, acc_sh.at[pl.ds(r0, RPT)])
    pltpu.sync_copy(ones_hbm, ones_v)
    pltpu.sync_copy(dst_hbm.at[pl.ds(w * NCHUNK, NCHUNK)], dst_v)
    plsc.subcore_barrier()

    def body(j, carry):
        pltpu.sync_copy(ones_v, acc_sh.at[dst_v.at[j]], add=True)
        return carry

    lax.fori_loop(0, NCHUNK, body, 0)
    plsc.subcore_barrier()
    pltpu.sync_copy(acc_sh.at[pl.ds(r0, RPT)], stage_v)
    pltpu.sync_copy(stage_v, out_hbm.at[pl.ds(c * NPAD + r0, RPT)])


def _make_agg(F):
    def body_fn(y_hbm, src_hbm, dst_hbm, zeros_hbm, out_hbm,
                src_v, dst_v, rows_v, rows2_v, zbuf_v, acc_sh, gsem, gsem2):
        c = lax.axis_index("c")
        s = lax.axis_index("s")
        w = c * NS + s
        r0 = s * RPT
        pltpu.sync_copy(zeros_hbm.at[pl.ds(0, ZCH)], zbuf_v)

        def zbody(k, carry):
            pltpu.sync_copy(zbuf_v, acc_sh.at[pl.ds(r0 + k * ZCH, ZCH)])
            return carry

        lax.fori_loop(0, NZB, zbody, 0)
        pltpu.sync_copy(src_hbm.at[pl.ds(w * NCHUNK, NCHUNK)], src_v)
        pltpu.sync_copy(dst_hbm.at[pl.ds(w * NCHUNK, NCHUNK)], dst_v)
        plsc.subcore_barrier()

        pltpu.async_copy(y_hbm.at[src_v.at[0]], rows_v, gsem)
        pltpu.async_copy(y_hbm.at[src_v.at[1]], rows2_v, gsem2)

        def body(i, carry):
            j0 = 2 * i
            pltpu.make_async_copy(y_hbm.at[src_v.at[j0]], rows_v, gsem).wait()
            pltpu.sync_copy(rows_v, acc_sh.at[dst_v.at[j0]], add=True)

            @pl.when(i + 1 < NCHUNK // 2)
            def _():
                pltpu.async_copy(y_hbm.at[src_v.at[j0 + 2]], rows_v, gsem)

            pltpu.make_async_copy(y_hbm.at[src_v.at[j0 + 1]], rows2_v, gsem2).wait()
            pltpu.sync_copy(rows2_v, acc_sh.at[dst_v.at[j0 + 1]], add=True)

            @pl.when(i + 1 < NCHUNK // 2)
            def _():
                pltpu.async_copy(y_hbm.at[src_v.at[j0 + 3]], rows2_v, gsem2)

            return carry

        lax.fori_loop(0, NCHUNK // 2, body, 0)
        plsc.subcore_barrier()

        def wbody(k, carry):
            pltpu.sync_copy(acc_sh.at[pl.ds(r0 + k * ZCH, ZCH)], zbuf_v)
            pltpu.sync_copy(zbuf_v, out_hbm.at[pl.ds(c * NPAD + r0 + k * ZCH, ZCH)])
            return carry

        lax.fori_loop(0, NZB, wbody, 0)

    return pl.kernel(
        body_fn,
        out_type=jax.ShapeDtypeStruct((NC * NPAD, F), jnp.float32),
        mesh=_sc_mesh,
        scratch_types=[
            pltpu.VMEM((NCHUNK, CH), jnp.int32),
            pltpu.VMEM((NCHUNK, CH), jnp.int32),
            pltpu.VMEM((CH, F), jnp.float32),
            pltpu.VMEM((CH, F), jnp.float32),
            pltpu.VMEM((ZCH, F), jnp.float32),
            pltpu.VMEM_SHARED((NPAD, F), jnp.float32),
            pltpu.SemaphoreType.DMA,
            pltpu.SemaphoreType.DMA,
        ],
        compiler_params=pltpu.CompilerParams(use_tc_tiling_on_sc=False),
    )


_sc_agg16 = _make_agg(H)
_sc_agg8 = _make_agg(8)


def _dis_of(deg_block):
    return lax.rsqrt(deg_block[:, 0] + deg_block[:, 1] + 1.0)


def _mm_body(x_ref, w_ref, deg_ref, y_ref):
    dis = _dis_of(deg_ref[...])
    xw = jnp.dot(x_ref[...], w_ref[...], preferred_element_type=jnp.float32)
    y_ref[...] = xw * dis[:, None]


def _mid_body(s1a_ref, s1b_ref, y1_ref, deg_ref, b1_ref, w2_ref, y2_ref):
    dis = _dis_of(deg_ref[...])
    pre = dis[:, None] * (s1a_ref[...] + s1b_ref[...] + y1_ref[...]) + b1_ref[...]
    h1 = jnp.maximum(pre, 0.0)
    z2 = jnp.dot(h1, w2_ref[...], preferred_element_type=jnp.float32)
    y2_ref[...] = z2 * dis[:, None]


def _fin_body(s2a_ref, s2b_ref, y2_ref, deg_ref, b2_ref, batch_ref,
              ls_ref, ht_ref):
    i = pl.program_id(0)
    dis = _dis_of(deg_ref[...])
    x2 = dis[:, None] * (s2a_ref[...] + s2b_ref[...] + y2_ref[...]) + b2_ref[...]
    neg = jnp.float32(-jnp.inf)
    col = lax.broadcasted_iota(jnp.int32, (BM, 8), 1)
    valid = col < C
    x2m = jnp.where(valid, x2, neg)
    m = jnp.max(x2m, axis=1, keepdims=True)
    ex = jnp.where(valid, jnp.exp(x2 - m), 0.0)
    lse = jnp.log(jnp.sum(ex, axis=1, keepdims=True)) + m
    ls_ref[...] = lax.slice(x2 - lse, (0, 0), (BM, C))

    b = batch_ref[...]
    gid = lax.broadcasted_iota(jnp.int32, (BM, G), 1)
    mask = b == gid

    @pl.when(i == 0)
    def _():
        ht_ref[...] = jnp.full((C, G), neg, jnp.float32)

    rows = []
    for cc in range(C):
        xc = lax.slice(x2, (0, cc), (BM, cc + 1))
        rows.append(jnp.max(jnp.where(mask, xc, neg), axis=0, keepdims=True))
    ht_ref[...] = jnp.maximum(ht_ref[...], jnp.concatenate(rows, axis=0))


_row_spec = pl.BlockSpec((BM, H), lambda i: (i, 0))
_deg_spec = pl.BlockSpec((BM, 2), lambda i: (i, 0))

_tc_mm = pl.pallas_call(
    _mm_body,
    grid=(NBLK,),
    in_specs=[
        pl.BlockSpec((BM, DIN), lambda i: (i, 0)),
        pl.BlockSpec((DIN, H), lambda i: (0, 0)),
        _deg_spec,
    ],
    out_specs=_row_spec,
    out_shape=jax.ShapeDtypeStruct((N, H), jnp.float32),
)

_row8_spec = pl.BlockSpec((BM, 8), lambda i: (i, 0))

_tc_mid = pl.pallas_call(
    _mid_body,
    grid=(NBLK,),
    in_specs=[
        _row_spec, _row_spec, _row_spec, _deg_spec,
        pl.BlockSpec((1, H), lambda i: (0, 0)),
        pl.BlockSpec((H, 8), lambda i: (0, 0)),
    ],
    out_specs=_row8_spec,
    out_shape=jax.ShapeDtypeStruct((N, 8), jnp.float32),
)

_tc_fin = pl.pallas_call(
    _fin_body,
    grid=(NBLK,),
    in_specs=[
        _row8_spec, _row8_spec, _row8_spec, _deg_spec,
        pl.BlockSpec((1, 8), lambda i: (0, 0)),
        pl.BlockSpec((BM, 1), lambda i: (i, 0)),
    ],
    out_specs=(
        pl.BlockSpec((BM, C), lambda i: (i, 0)),
        pl.BlockSpec((C, G), lambda i: (0, 0)),
    ),
    out_shape=(
        jax.ShapeDtypeStruct((N, C), jnp.float32),
        jax.ShapeDtypeStruct((C, G), jnp.float32),
    ),
)


def kernel(x, edge_index, batch, W1, b1, W2, b2):
    ei = edge_index.astype(jnp.int32)
    npad = EPAD - E
    ar = jnp.arange(npad, dtype=jnp.int32)
    pad_src = (ar * 97) % N
    pad_dst = N + (ar % (NPAD - N))
    srcp = jnp.concatenate([ei[0], pad_src]).reshape(NW * NCHUNK, CH)
    dstp = jnp.concatenate([ei[1], pad_dst]).reshape(NW * NCHUNK, CH)

    zeros16 = jnp.zeros((NPAD, H), jnp.float32)
    zeros8 = jnp.zeros((NPAD, 8), jnp.float32)
    zeros1 = jnp.zeros((NPAD,), jnp.float32)
    ones = jnp.ones((CH,), jnp.float32)

    deg2 = _sc_deg(dstp, zeros1, ones)
    degT = deg2.reshape(NC, NPAD).T

    y1 = _tc_mm(x, W1, degT[:N])
    s1 = _sc_agg16(y1, srcp, dstp, zeros16)

    W2p = jnp.zeros((H, 8), jnp.float32).at[:, :C].set(W2)
    b1r = b1.reshape(1, H)
    y2 = _tc_mid(s1[:N], s1[NPAD:NPAD + N], y1, degT[:N], b1r, W2p)

    s2 = _sc_agg8(y2, srcp, dstp, zeros8)
    b2p = jnp.pad(b2, (0, 8 - C)).reshape(1, 8)
    batch2 = batch.astype(jnp.int32).reshape(N, 1)
    ls, ht = _tc_fin(s2[:N], s2[NPAD:NPAD + N], y2, degT[:N], b2p, batch2)
    return (ht.T, ls)


# speedup vs baseline: 35.4747x; 1.4195x over previous
import functools
import jax
import jax.numpy as jnp
from jax import lax
from jax.experimental import pallas as pl
from jax.experimental.pallas import tpu as pltpu
from jax.experimental.pallas import tpu_sc as plsc

N = 50000
NPAD = 50048
E = 800000
DIN = 1433
H = 16
C = 7
G = 64
NC = 2
NS = 16
NW = NC * NS
CH = 128
NCHUNK = 200
EW = CH * NCHUNK
EPAD = EW * NW
BM = 1024
NBLK = -(-N // BM)
RPT = NPAD // NS
ZCH = 136
NZB = RPT // ZCH

_sc_mesh = plsc.VectorSubcoreMesh(core_axis_name="c", subcore_axis_name="s")


@functools.partial(
    pl.kernel,
    out_type=jax.ShapeDtypeStruct((NC * NPAD,), jnp.float32),
    mesh=_sc_mesh,
    scratch_types=[
        pltpu.VMEM((NCHUNK, CH), jnp.int32),
        pltpu.VMEM((CH,), jnp.float32),
        pltpu.VMEM((RPT,), jnp.float32),
        pltpu.VMEM_SHARED((NPAD,), jnp.float32),
    ],
    compiler_params=pltpu.CompilerParams(use_tc_tiling_on_sc=False),
)
def _sc_deg(dst_hbm, zeros_hbm, ones_hbm, out_hbm, dst_v, ones_v, stage_v,
            acc_sh):
    c = lax.axis_index("c")
    s = lax.axis_index("s")
    w = c * NS + s
    r0 = s * RPT
    pltpu.sync_copy(zeros_hbm.at[pl.ds(r0, RPT)], stage_v)
    pltpu.sync_copy(stage_v, acc_sh.at[pl.ds(r0, RPT)])
    pltpu.sync_copy(ones_hbm, ones_v)
    pltpu.sync_copy(dst_hbm.at[pl.ds(w * NCHUNK, NCHUNK)], dst_v)
    plsc.subcore_barrier()

    def body(j, carry):
        pltpu.sync_copy(ones_v, acc_sh.at[dst_v.at[j]], add=True)
        return carry

    lax.fori_loop(0, NCHUNK, body, 0)
    plsc.subcore_barrier()
    pltpu.sync_copy(acc_sh.at[pl.ds(r0, RPT)], stage_v)
    pltpu.sync_copy(stage_v, out_hbm.at[pl.ds(c * NPAD + r0, RPT)])


def _make_agg(F):
    def body_fn(y_hbm, src_hbm, dst_hbm, zeros_hbm, out0_hbm, out1_hbm,
                src_v, dst_v, rows_v, rows2_v, zbuf_v, acc_sh, gsem, gsem2):
        c = lax.axis_index("c")
        s = lax.axis_index("s")
        w = c * NS + s
        r0 = s * RPT
        pltpu.sync_copy(zeros_hbm.at[pl.ds(0, ZCH)], zbuf_v)

        def zbody(k, carry):
            pltpu.sync_copy(zbuf_v, acc_sh.at[pl.ds(r0 + k * ZCH, ZCH)])
            return carry

        lax.fori_loop(0, NZB, zbody, 0)
        pltpu.sync_copy(src_hbm.at[pl.ds(w * NCHUNK, NCHUNK)], src_v)
        pltpu.sync_copy(dst_hbm.at[pl.ds(w * NCHUNK, NCHUNK)], dst_v)
        plsc.subcore_barrier()

        pltpu.async_copy(y_hbm.at[src_v.at[0]], rows_v, gsem)
        pltpu.async_copy(y_hbm.at[src_v.at[1]], rows2_v, gsem2)

        def body(i, carry):
            j0 = 2 * i
            pltpu.make_async_copy(y_hbm.at[src_v.at[j0]], rows_v, gsem).wait()
            pltpu.sync_copy(rows_v, acc_sh.at[dst_v.at[j0]], add=True)

            @pl.when(i + 1 < NCHUNK // 2)
            def _():
                pltpu.async_copy(y_hbm.at[src_v.at[j0 + 2]], rows_v, gsem)

            pltpu.make_async_copy(y_hbm.at[src_v.at[j0 + 1]], rows2_v, gsem2).wait()
            pltpu.sync_copy(rows2_v, acc_sh.at[dst_v.at[j0 + 1]], add=True)

            @pl.when(i + 1 < NCHUNK // 2)
            def _():
                pltpu.async_copy(y_hbm.at[src_v.at[j0 + 3]], rows2_v, gsem2)

            return carry

        lax.fori_loop(0, NCHUNK // 2, body, 0)
        plsc.subcore_barrier()

        @pl.when(c == 0)
        def _():
            def wbody(k, carry):
                pltpu.sync_copy(acc_sh.at[pl.ds(r0 + k * ZCH, ZCH)], zbuf_v)
                pltpu.sync_copy(zbuf_v, out0_hbm.at[pl.ds(r0 + k * ZCH, ZCH)])
                return carry

            lax.fori_loop(0, NZB, wbody, 0)

        @pl.when(c == 1)
        def _():
            def wbody(k, carry):
                pltpu.sync_copy(acc_sh.at[pl.ds(r0 + k * ZCH, ZCH)], zbuf_v)
                pltpu.sync_copy(zbuf_v, out1_hbm.at[pl.ds(r0 + k * ZCH, ZCH)])
                return carry

            lax.fori_loop(0, NZB, wbody, 0)

    return pl.kernel(
        body_fn,
        out_type=(jax.ShapeDtypeStruct((NPAD, F), jnp.float32),
                  jax.ShapeDtypeStruct((NPAD, F), jnp.float32)),
        mesh=_sc_mesh,
        scratch_types=[
            pltpu.VMEM((NCHUNK, CH), jnp.int32),
            pltpu.VMEM((NCHUNK, CH), jnp.int32),
            pltpu.VMEM((CH, F), jnp.float32),
            pltpu.VMEM((CH, F), jnp.float32),
            pltpu.VMEM((ZCH, F), jnp.float32),
            pltpu.VMEM_SHARED((NPAD, F), jnp.float32),
            pltpu.SemaphoreType.DMA,
            pltpu.SemaphoreType.DMA,
        ],
        compiler_params=pltpu.CompilerParams(use_tc_tiling_on_sc=False),
    )


_sc_agg16 = _make_agg(H)
_sc_agg8 = _make_agg(8)


def _dis_of(deg_block):
    return lax.rsqrt(deg_block[:, 0] + deg_block[:, 1] + 1.0)


def _mm_body(xt_ref, w_ref, deg_ref, y_ref):
    dis = _dis_of(deg_ref[...])
    xw = lax.dot_general(xt_ref[...], w_ref[...], (((0,), (0,)), ((), ())),
                         preferred_element_type=jnp.float32)
    y_ref[...] = xw * dis[:, None]


def _mid_body(s1a_ref, s1b_ref, y1_ref, deg_ref, b1_ref, w2_ref, y2_ref):
    dis = _dis_of(deg_ref[...])
    pre = dis[:, None] * (s1a_ref[...] + s1b_ref[...] + y1_ref[...]) + b1_ref[...]
    h1 = jnp.maximum(pre, 0.0)
    z2 = jnp.dot(h1, w2_ref[...], preferred_element_type=jnp.float32)
    y2_ref[...] = z2 * dis[:, None]


def _fin_body(s2a_ref, s2b_ref, y2_ref, deg_ref, b2_ref, batch_ref,
              ls_ref, ht_ref):
    i = pl.program_id(0)
    dis = _dis_of(deg_ref[...])
    x2 = dis[:, None] * (s2a_ref[...] + s2b_ref[...] + y2_ref[...]) + b2_ref[...]
    neg = jnp.float32(-jnp.inf)
    col = lax.broadcasted_iota(jnp.int32, (BM, 8), 1)
    valid = col < C
    x2m = jnp.where(valid, x2, neg)
    m = jnp.max(x2m, axis=1, keepdims=True)
    ex = jnp.where(valid, jnp.exp(x2 - m), 0.0)
    lse = jnp.log(jnp.sum(ex, axis=1, keepdims=True)) + m
    ls_ref[...] = lax.slice(x2 - lse, (0, 0), (BM, C))

    b = batch_ref[...]
    gid = lax.broadcasted_iota(jnp.int32, (BM, G), 1)
    row = lax.broadcasted_iota(jnp.int32, (BM, G), 0)
    mask = (b == gid) & (row + i * BM < N)

    @pl.when(i == 0)
    def _():
        ht_ref[...] = jnp.full((C, G), neg, jnp.float32)

    rows = []
    for cc in range(C):
        xc = lax.slice(x2, (0, cc), (BM, cc + 1))
        rows.append(jnp.max(jnp.where(mask, xc, neg), axis=0, keepdims=True))
    ht_ref[...] = jnp.maximum(ht_ref[...], jnp.concatenate(rows, axis=0))


_row_spec = pl.BlockSpec((BM, H), lambda i: (i, 0))
_deg_spec = pl.BlockSpec((BM, 2), lambda i: (i, 0))

_tc_mm = pl.pallas_call(
    _mm_body,
    grid=(NBLK,),
    in_specs=[
        pl.BlockSpec((DIN, BM), lambda i: (0, i)),
        pl.BlockSpec((DIN, H), lambda i: (0, 0)),
        _deg_spec,
    ],
    out_specs=_row_spec,
    out_shape=jax.ShapeDtypeStruct((N, H), jnp.float32),
)

_row8_spec = pl.BlockSpec((BM, 8), lambda i: (i, 0))

_tc_mid = pl.pallas_call(
    _mid_body,
    grid=(NBLK,),
    in_specs=[
        _row_spec, _row_spec, _row_spec, _deg_spec,
        pl.BlockSpec((1, H), lambda i: (0, 0)),
        pl.BlockSpec((H, 8), lambda i: (0, 0)),
    ],
    out_specs=_row8_spec,
    out_shape=jax.ShapeDtypeStruct((N, 8), jnp.float32),
)

_tc_fin = pl.pallas_call(
    _fin_body,
    grid=(NBLK,),
    in_specs=[
        _row8_spec, _row8_spec, _row8_spec, _deg_spec,
        pl.BlockSpec((1, 8), lambda i: (0, 0)),
        pl.BlockSpec((BM, 1), lambda i: (i, 0)),
    ],
    out_specs=(
        pl.BlockSpec((BM, C), lambda i: (i, 0)),
        pl.BlockSpec((C, G), lambda i: (0, 0)),
    ),
    out_shape=(
        jax.ShapeDtypeStruct((N, C), jnp.float32),
        jax.ShapeDtypeStruct((C, G), jnp.float32),
    ),
)


def kernel(x, edge_index, batch, W1, b1, W2, b2):
    ei = edge_index.astype(jnp.int32)
    npad = EPAD - E
    ar = jnp.arange(npad, dtype=jnp.int32)
    pad_src = (ar * 97) % N
    pad_dst = N + (ar % (NPAD - N))
    srcp = jnp.concatenate([ei[0], pad_src]).reshape(NW * NCHUNK, CH)
    dstp = jnp.concatenate([ei[1], pad_dst]).reshape(NW * NCHUNK, CH)

    zeros16 = jnp.zeros((NPAD, H), jnp.float32)
    zeros8 = jnp.zeros((NPAD, 8), jnp.float32)
    zeros1 = jnp.zeros((NPAD,), jnp.float32)
    ones = jnp.ones((CH,), jnp.float32)

    deg2 = _sc_deg(dstp, zeros1, ones)
    degT = deg2.reshape(NC, NPAD).T

    y1 = _tc_mm(x.T, W1, degT)
    s1a, s1b = _sc_agg16(y1, srcp, dstp, zeros16)

    W2p = jnp.zeros((H, 8), jnp.float32).at[:, :C].set(W2)
    b1r = b1.reshape(1, H)
    y2 = _tc_mid(s1a, s1b, y1, degT, b1r, W2p)

    s2a, s2b = _sc_agg8(y2, srcp, dstp, zeros8)
    b2p = jnp.pad(b2, (0, 8 - C)).reshape(1, 8)
    batch2 = batch.astype(jnp.int32).reshape(N, 1)
    ls, ht = _tc_fin(s2a, s2b, y2, degT, b2p, batch2)
    return (ht.T, ls)


# speedup vs baseline: 37.1376x; 1.0469x over previous
import functools
import jax
import jax.numpy as jnp
from jax import lax
from jax.experimental import pallas as pl
from jax.experimental.pallas import tpu as pltpu
from jax.experimental.pallas import tpu_sc as plsc

N = 50000
NPAD = 50048
E = 800000
DIN = 1433
H = 16
C = 7
G = 64
NC = 2
NS = 16
NW = NC * NS
CH = 128
NCHUNK = 200
EW = CH * NCHUNK
EPAD = EW * NW
BM = 2048
NBLK = -(-N // BM)
RPT = NPAD // NS
ZCH = 136
NZB = RPT // ZCH

_sc_mesh = plsc.VectorSubcoreMesh(core_axis_name="c", subcore_axis_name="s")


@functools.partial(
    pl.kernel,
    out_type=jax.ShapeDtypeStruct((NC * NPAD,), jnp.float32),
    mesh=_sc_mesh,
    scratch_types=[
        pltpu.VMEM((NCHUNK, CH), jnp.int32),
        pltpu.VMEM((CH,), jnp.float32),
        pltpu.VMEM((RPT,), jnp.float32),
        pltpu.VMEM_SHARED((NPAD,), jnp.float32),
    ],
    compiler_params=pltpu.CompilerParams(use_tc_tiling_on_sc=False),
)
def _sc_deg(dst_hbm, zeros_hbm, ones_hbm, out_hbm, dst_v, ones_v, stage_v,
            acc_sh):
    c = lax.axis_index("c")
    s = lax.axis_index("s")
    w = c * NS + s
    r0 = s * RPT
    pltpu.sync_copy(zeros_hbm.at[pl.ds(r0, RPT)], stage_v)
    pltpu.sync_copy(stage_v, acc_sh.at[pl.ds(r0, RPT)])
    pltpu.sync_copy(ones_hbm, ones_v)
    pltpu.sync_copy(dst_hbm.at[pl.ds(w * NCHUNK, NCHUNK)], dst_v)
    plsc.subcore_barrier()

    def body(j, carry):
        pltpu.sync_copy(ones_v, acc_sh.at[dst_v.at[j]], add=True)
        return carry

    lax.fori_loop(0, NCHUNK, body, 0)
    plsc.subcore_barrier()
    pltpu.sync_copy(acc_sh.at[pl.ds(r0, RPT)], stage_v)
    pltpu.sync_copy(stage_v, out_hbm.at[pl.ds(c * NPAD + r0, RPT)])


def _make_agg(F):
    def body_fn(y_hbm, src_hbm, dst_hbm, zeros_hbm, out0_hbm, out1_hbm,
                src_v, dst_v, rows_v, rows2_v, zbuf_v, acc_sh, gsem, gsem2):
        c = lax.axis_index("c")
        s = lax.axis_index("s")
        w = c * NS + s
        r0 = s * RPT
        pltpu.sync_copy(zeros_hbm.at[pl.ds(0, ZCH)], zbuf_v)

        def zbody(k, carry):
            pltpu.sync_copy(zbuf_v, acc_sh.at[pl.ds(r0 + k * ZCH, ZCH)])
            return carry

        lax.fori_loop(0, NZB, zbody, 0)
        pltpu.sync_copy(src_hbm.at[pl.ds(w * NCHUNK, NCHUNK)], src_v)
        pltpu.sync_copy(dst_hbm.at[pl.ds(w * NCHUNK, NCHUNK)], dst_v)
        plsc.subcore_barrier()

        pltpu.async_copy(y_hbm.at[src_v.at[0]], rows_v, gsem)
        pltpu.async_copy(y_hbm.at[src_v.at[1]], rows2_v, gsem2)

        def body(i, carry):
            j0 = 2 * i
            pltpu.make_async_copy(y_hbm.at[src_v.at[j0]], rows_v, gsem).wait()
            pltpu.sync_copy(rows_v, acc_sh.at[dst_v.at[j0]], add=True)

            @pl.when(i + 1 < NCHUNK // 2)
            def _():
                pltpu.async_copy(y_hbm.at[src_v.at[j0 + 2]], rows_v, gsem)

            pltpu.make_async_copy(y_hbm.at[src_v.at[j0 + 1]], rows2_v, gsem2).wait()
            pltpu.sync_copy(rows2_v, acc_sh.at[dst_v.at[j0 + 1]], add=True)

            @pl.when(i + 1 < NCHUNK // 2)
            def _():
                pltpu.async_copy(y_hbm.at[src_v.at[j0 + 3]], rows2_v, gsem2)

            return carry

        lax.fori_loop(0, NCHUNK // 2, body, 0)
        plsc.subcore_barrier()

        @pl.when(c == 0)
        def _():
            def wbody(k, carry):
                pltpu.sync_copy(acc_sh.at[pl.ds(r0 + k * ZCH, ZCH)], zbuf_v)
                pltpu.sync_copy(zbuf_v, out0_hbm.at[pl.ds(r0 + k * ZCH, ZCH)])
                return carry

            lax.fori_loop(0, NZB, wbody, 0)

        @pl.when(c == 1)
        def _():
            def wbody(k, carry):
                pltpu.sync_copy(acc_sh.at[pl.ds(r0 + k * ZCH, ZCH)], zbuf_v)
                pltpu.sync_copy(zbuf_v, out1_hbm.at[pl.ds(r0 + k * ZCH, ZCH)])
                return carry

            lax.fori_loop(0, NZB, wbody, 0)

    return pl.kernel(
        body_fn,
        out_type=(jax.ShapeDtypeStruct((NPAD, F), jnp.float32),
                  jax.ShapeDtypeStruct((NPAD, F), jnp.float32)),
        mesh=_sc_mesh,
        scratch_types=[
            pltpu.VMEM((NCHUNK, CH), jnp.int32),
            pltpu.VMEM((NCHUNK, CH), jnp.int32),
            pltpu.VMEM((CH, F), jnp.float32),
            pltpu.VMEM((CH, F), jnp.float32),
            pltpu.VMEM((ZCH, F), jnp.float32),
            pltpu.VMEM_SHARED((NPAD, F), jnp.float32),
            pltpu.SemaphoreType.DMA,
            pltpu.SemaphoreType.DMA,
        ],
        compiler_params=pltpu.CompilerParams(use_tc_tiling_on_sc=False),
    )


_sc_agg16 = _make_agg(H)
_sc_agg8 = _make_agg(8)


def _dis_of(deg_block):
    return lax.rsqrt(deg_block[:, 0] + deg_block[:, 1] + 1.0)


def _mm_body(xt_ref, w_ref, deg_ref, y_ref):
    dis = _dis_of(deg_ref[...])
    xw = lax.dot_general(xt_ref[...], w_ref[...], (((0,), (0,)), ((), ())),
                         preferred_element_type=jnp.float32)
    y_ref[...] = xw * dis[:, None]


def _mid_body(s1a_ref, s1b_ref, y1_ref, deg_ref, b1_ref, w2_ref, y2_ref):
    dis = _dis_of(deg_ref[...])
    pre = dis[:, None] * (s1a_ref[...] + s1b_ref[...] + y1_ref[...]) + b1_ref[...]
    h1 = jnp.maximum(pre, 0.0)
    z2 = jnp.dot(h1, w2_ref[...], preferred_element_type=jnp.float32)
    y2_ref[...] = z2 * dis[:, None]


def _fin_body(s2a_ref, s2b_ref, y2_ref, deg_ref, b2_ref, batch_ref,
              ls_ref, ht_ref):
    i = pl.program_id(0)
    dis = _dis_of(deg_ref[...])
    x2 = dis[:, None] * (s2a_ref[...] + s2b_ref[...] + y2_ref[...]) + b2_ref[...]
    neg = jnp.float32(-jnp.inf)
    col = lax.broadcasted_iota(jnp.int32, (BM, 8), 1)
    valid = col < C
    x2m = jnp.where(valid, x2, neg)
    m = jnp.max(x2m, axis=1, keepdims=True)
    ex = jnp.where(valid, jnp.exp(x2 - m), 0.0)
    lse = jnp.log(jnp.sum(ex, axis=1, keepdims=True)) + m
    ls_ref[...] = lax.slice(x2 - lse, (0, 0), (BM, C))

    b = batch_ref[...]
    gid = lax.broadcasted_iota(jnp.int32, (BM, G), 1)
    row = lax.broadcasted_iota(jnp.int32, (BM, G), 0)
    mask = (b == gid) & (row + i * BM < N)

    @pl.when(i == 0)
    def _():
        ht_ref[...] = jnp.full((C, G), neg, jnp.float32)

    rows = []
    for cc in range(C):
        xc = lax.slice(x2, (0, cc), (BM, cc + 1))
        rows.append(jnp.max(jnp.where(mask, xc, neg), axis=0, keepdims=True))
    ht_ref[...] = jnp.maximum(ht_ref[...], jnp.concatenate(rows, axis=0))


_row_spec = pl.BlockSpec((BM, H), lambda i: (i, 0))
_deg_spec = pl.BlockSpec((BM, 2), lambda i: (i, 0))

_tc_mm = pl.pallas_call(
    _mm_body,
    grid=(NBLK,),
    in_specs=[
        pl.BlockSpec((DIN, BM), lambda i: (0, i)),
        pl.BlockSpec((DIN, H), lambda i: (0, 0)),
        _deg_spec,
    ],
    out_specs=_row_spec,
    out_shape=jax.ShapeDtypeStruct((N, H), jnp.float32),
)

_row8_spec = pl.BlockSpec((BM, 8), lambda i: (i, 0))

_tc_mid = pl.pallas_call(
    _mid_body,
    grid=(NBLK,),
    in_specs=[
        _row_spec, _row_spec, _row_spec, _deg_spec,
        pl.BlockSpec((1, H), lambda i: (0, 0)),
        pl.BlockSpec((H, 8), lambda i: (0, 0)),
    ],
    out_specs=_row8_spec,
    out_shape=jax.ShapeDtypeStruct((N, 8), jnp.float32),
)

_tc_fin = pl.pallas_call(
    _fin_body,
    grid=(NBLK,),
    in_specs=[
        _row8_spec, _row8_spec, _row8_spec, _deg_spec,
        pl.BlockSpec((1, 8), lambda i: (0, 0)),
        pl.BlockSpec((BM, 1), lambda i: (i, 0)),
    ],
    out_specs=(
        pl.BlockSpec((BM, C), lambda i: (i, 0)),
        pl.BlockSpec((C, G), lambda i: (0, 0)),
    ),
    out_shape=(
        jax.ShapeDtypeStruct((N, C), jnp.float32),
        jax.ShapeDtypeStruct((C, G), jnp.float32),
    ),
)


def kernel(x, edge_index, batch, W1, b1, W2, b2):
    ei = edge_index.astype(jnp.int32)
    npad = EPAD - E
    ar = jnp.arange(npad, dtype=jnp.int32)
    pad_src = (ar * 97) % N
    pad_dst = N + (ar % (NPAD - N))
    srcp = jnp.concatenate([ei[0], pad_src]).reshape(NW * NCHUNK, CH)
    dstp = jnp.concatenate([ei[1], pad_dst]).reshape(NW * NCHUNK, CH)

    zeros16 = jnp.zeros((NPAD, H), jnp.float32)
    zeros8 = jnp.zeros((NPAD, 8), jnp.float32)
    zeros1 = jnp.zeros((NPAD,), jnp.float32)
    ones = jnp.ones((CH,), jnp.float32)

    deg2 = _sc_deg(dstp, zeros1, ones)
    degT = deg2.reshape(NC, NPAD).T

    y1 = _tc_mm(x.T, W1, degT)
    s1a, s1b = _sc_agg16(y1, srcp, dstp, zeros16)

    W2p = jnp.zeros((H, 8), jnp.float32).at[:, :C].set(W2)
    b1r = b1.reshape(1, H)
    y2 = _tc_mid(s1a, s1b, y1, degT, b1r, W2p)

    s2a, s2b = _sc_agg8(y2, srcp, dstp, zeros8)
    b2p = jnp.pad(b2, (0, 8 - C)).reshape(1, 8)
    batch2 = batch.astype(jnp.int32).reshape(N, 1)
    ls, ht = _tc_fin(s2a, s2b, y2, degT, b2p, batch2)
    return (ht.T, ls)
